# Initial kernel scaffold; baseline (speedup 1.0000x reference)
import jax
import jax.numpy as jnp
from jax.experimental import pallas as pl


def kernel(x, edge_index, batch, params):
    raise NotImplementedError("write your pallas kernel here")


# speedup vs baseline: 1.0059x; 1.0059x over previous
import functools

import jax
import jax.numpy as jnp
from jax.experimental import pallas as pl
from jax.experimental.pallas import tpu as pltpu


def _mm_kernel(x_ref, w_ref, b_ref, o_ref):
    o_ref[...] = (
        jnp.dot(
            x_ref[...].astype(jnp.bfloat16),
            w_ref[...].astype(jnp.bfloat16),
            preferred_element_type=jnp.float32,
        )
        + b_ref[...]
    )


def _matmul_bias(x, w, b):
    m, k = x.shape
    k2, n = w.shape
    assert k == k2
    return pl.pallas_call(
        _mm_kernel,
        out_shape=jax.ShapeDtypeStruct((m, n), jnp.float32),
    )(x, w, b[None, :])


def _seg_sum(d, i, n):
    return jax.ops.segment_sum(d, i, num_segments=n)


def _seg_max(d, i, n):
    return jax.ops.segment_max(d, i, num_segments=n)


def _gatv2(x, src, dst, N, p):
    xl = _matmul_bias(x, p['Wl'], p['bl'])
    xr = _matmul_bias(x, p['Wr'], p['br'])
    z = jax.nn.leaky_relu(xl[src] + xr[dst], 0.2)
    logit = z @ p['att']
    m = _seg_max(logit, dst, N)
    m = jnp.where(jnp.isfinite(m), m, 0.0)
    e = jnp.exp(logit - m[dst])
    s = _seg_sum(e, dst, N)
    alpha = e / (s[dst] + 1e-16)
    return _seg_sum(xl[src] * alpha[:, None], dst, N) + p['bias']


def _graph_norm(x, batch, B, cnt, p):
    mean = _seg_sum(x, batch, B) / cnt[:, None]
    out = x - mean[batch] * p['mean_scale']
    var = _seg_sum(out * out, batch, B) / cnt[:, None]
    return out / jnp.sqrt(var + 1e-5)[batch] * p['weight'] + p['bias']


def _arma(x, src, dst, N, norm_w, p):
    h = x @ p['W']
    agg = _seg_sum(h[src] * norm_w[:, None], dst, N)
    return jax.nn.relu(agg + x @ p['Wr'] + p['b'])


def kernel(x, edge_index, batch, params):
    src, dst = edge_index[0], edge_index[1]
    N = x.shape[0]
    B = 50
    k = 12
    deg = _seg_sum(jnp.ones(src.shape, jnp.float32), dst, N)
    dinv = jnp.where(deg > 0, 1.0 / jnp.sqrt(jnp.maximum(deg, 1e-12)), 0.0)
    norm_w = dinv[src] * dinv[dst]
    cnt_i = jnp.bincount(batch, length=B)
    cnt = jnp.maximum(cnt_i.astype(jnp.float32), 1.0)

    h = x
    for i in range(3):
        h = _gatv2(h, src, dst, N, params['gat%d' % i])
        h = jax.nn.elu(h)
        h = _graph_norm(h, batch, B, cnt, params['gatn%d' % i])
    g = x
    for i in range(3):
        g = _arma(g, src, dst, N, norm_w, params['arma%d' % i])
        g = jax.nn.elu(g)
        g = _graph_norm(g, batch, B, cnt, params['arman%d' % i])
    hg = jnp.concatenate([h, g], axis=1)
    xmax = _seg_max(hg, batch, B)
    xmax = jnp.where(jnp.isfinite(xmax), xmax, 0.0)
    xsum = _seg_sum(hg, batch, B)
    xmean = xsum / cnt[:, None]
    pooled = jnp.concatenate([xmax, xmean, xsum], axis=1)
    pooled = _matmul_bias(pooled, params['lin1']['W'], params['lin1']['b'])
    D = hg.shape[1]
    fill = jnp.min(hg) - 1.0
    key = hg[:, -1]
    idxs = jnp.arange(N)
    avail = jnp.ones((N,), dtype=bool)
    rows = []
    for _ in range(k):
        mk = jnp.where(avail, key, -jnp.inf)
        m = _seg_max(mk, batch, B)
        cand = avail & (key == m[batch])
        sel = jax.ops.segment_min(jnp.where(cand, idxs, N), batch, num_segments=B)
        has = sel < N
        safe = jnp.minimum(sel, N - 1)
        rows.append(jnp.where(has[:, None], hg[safe], fill))
        picked = jnp.zeros((N,), dtype=bool).at[safe].max(has)
        avail = avail & ~picked
    top = jnp.stack(rows, axis=1)
    top = jnp.where(top == fill, 0.0, top)
    agg = top.reshape(B, k * D)
    agg = _matmul_bias(agg, params['lin2']['W'], params['lin2']['b'])
    out = jnp.concatenate([pooled, agg], axis=1)
    return out @ params['lin3']['W']
